# baseline (speedup 1.0000x reference)
import functools

import jax
import jax.numpy as jnp
from jax import lax
from jax.experimental import pallas as pl
from jax.experimental.pallas import tpu as pltpu
from jax.experimental.pallas import tpu_sc as plsc

VOCAB = 27
EMB = 5
OUT = 27
PAD = 32
ROWS = 736
BATCH = 16384

NC = 2
NS = 16
NW = NC * NS
B_PER_W = BATCH // NW
CHUNK = 128
NCHUNK = B_PER_W // CHUNK
LANES = 16
PAR_PAD = 408


def _sc_body(par_hbm, xst_hbm, out_hbm, tab_hbm,
             par_v, x0_v, x1_v, idx_v, tabloc_v, rows_t_v, tabv_v,
             gsem, wsem):
    sid = lax.axis_index("s")
    wid = sid * NC + lax.axis_index("c")
    base = wid * B_PER_W

    x0c = pltpu.async_copy(xst_hbm.at[0, pl.ds(base, B_PER_W)], x0_v, gsem)
    x1c = pltpu.async_copy(xst_hbm.at[1, pl.ds(base, B_PER_W)], x1_v, gsem)
    pltpu.sync_copy(par_hbm, par_v)
    x0c.wait()
    x1c.wait()
    WOFF = VOCAB * EMB

    i16 = lax.iota(jnp.int32, LANES)
    hi_mask = (i16 + LANES) < OUT
    zro = x0_v[pl.ds(0, LANES)] >> 5

    def wrow(k):
        lo = par_v[pl.ds(WOFF + k * OUT, LANES)]
        hi = plsc.load_gather(
            par_v, [zro + i16 + (WOFF + k * OUT + LANES)], mask=hi_mask
        )
        return lo, hi

    w_lo = []
    w_hi = []
    for k in range(2 * EMB):
        lo, hi = wrow(k)
        w_lo.append(lo)
        w_hi.append(hi)

    def escal(row, k):
        return plsc.load_gather(par_v, [zro + (row * EMB + k)])

    for a in (sid, sid + NS):
        a_ok = a < VOCAB

        @pl.when(a_ok)
        def _build(a=a):
            t1_lo = jnp.zeros((LANES,), jnp.float32)
            t1_hi = jnp.zeros((LANES,), jnp.float32)
            for k in range(EMB):
                e = escal(a, k)
                ef = e.astype(jnp.float32)
                t1_lo = t1_lo + ef * w_lo[k]
                t1_hi = t1_hi + ef * w_hi[k]
            for b in range(VOCAB):
                r_lo = t1_lo
                r_hi = t1_hi
                for k in range(EMB):
                    e = escal(b, k)
                    ef = e.astype(jnp.float32)
                    r_lo = r_lo + ef * w_lo[EMB + k]
                    r_hi = r_hi + ef * w_hi[EMB + k]
                tabloc_v[b + 1, pl.ds(0, LANES)] = r_lo
                tabloc_v[b + 1, pl.ds(LANES, LANES)] = r_hi
            pltpu.sync_copy(
                tabloc_v.at[pl.ds(1, VOCAB), :],
                tab_hbm.at[pl.ds(a * VOCAB, VOCAB), :],
            )

    for j in range(B_PER_W // LANES):
        o = j * LANES
        idx_v[pl.ds(o, LANES)] = (
            x0_v[pl.ds(o, LANES)] * VOCAB + x1_v[pl.ds(o, LANES)]
        )
    plsc.subcore_barrier()
    pltpu.sync_copy(tab_hbm, tabv_v)
    for j in range(B_PER_W // LANES):
        idx16 = idx_v[pl.ds(j * LANES, LANES)]
        for c in range(OUT):
            rows_t_v[c, pl.ds(j * LANES, LANES)] = plsc.load_gather(
                tabv_v, [idx16, zro + c]
            )
    pltpu.sync_copy(rows_t_v, out_hbm.at[:, pl.ds(base, B_PER_W)])


@functools.lru_cache(maxsize=None)
def _make_kernel():
    return pl.kernel(
        _sc_body,
        out_type=(
            jax.ShapeDtypeStruct((PAD, BATCH), jnp.float32),
            jax.ShapeDtypeStruct((ROWS, PAD), jnp.float32),
        ),
        mesh=plsc.VectorSubcoreMesh(core_axis_name="c", subcore_axis_name="s"),
        compiler_params=pltpu.CompilerParams(
            needs_layout_passes=False, use_tc_tiling_on_sc=False
        ),
        scratch_types=[
            pltpu.VMEM((PAR_PAD,), jnp.float32),
            pltpu.VMEM((B_PER_W,), jnp.int32),
            pltpu.VMEM((B_PER_W,), jnp.int32),
            pltpu.VMEM((B_PER_W,), jnp.int32),
            pltpu.VMEM((VOCAB + 1, PAD), jnp.float32),
            pltpu.VMEM((PAD, B_PER_W), jnp.float32),
            pltpu.VMEM((ROWS, PAD), jnp.float32),
            pltpu.SemaphoreType.DMA,
            pltpu.SemaphoreType.DMA,
        ],
    )


def kernel(xs, embedding, W):
    par = jnp.concatenate([
        embedding.reshape(-1),
        W.reshape(-1),
        jnp.zeros((PAR_PAD - VOCAB * EMB - 2 * EMB * OUT,), jnp.float32),
    ])
    out_t, _ = _make_kernel()(par, xs.T)
    return out_t.T[:, :OUT]


# speedup vs baseline: 1.2421x; 1.2421x over previous
import functools

import jax
import jax.numpy as jnp
from jax import lax
from jax.experimental import pallas as pl
from jax.experimental.pallas import tpu as pltpu
from jax.experimental.pallas import tpu_sc as plsc

VOCAB = 27
EMB = 5
OUT = 27
PAD = 32
ROWS = 736
BATCH = 16384

NC = 2
NS = 16
NW = NC * NS
B_PER_W = BATCH // NW
CHUNK = 128
NCHUNK = B_PER_W // CHUNK
LANES = 16
PAR_PAD = 408


def _sc_body(par_hbm, xst_hbm, out_hbm,
             par_v, x0_v, x1_v, idx_v, tabloc_v, rows_v, tab_s,
             gsem, wsem):
    sid = lax.axis_index("s")
    wid = sid * NC + lax.axis_index("c")
    base = wid * B_PER_W

    x0c = pltpu.async_copy(xst_hbm.at[0, pl.ds(base, B_PER_W)], x0_v, gsem)
    x1c = pltpu.async_copy(xst_hbm.at[1, pl.ds(base, B_PER_W)], x1_v, gsem)
    pltpu.sync_copy(par_hbm, par_v)
    x0c.wait()
    x1c.wait()
    WOFF = VOCAB * EMB

    i16 = lax.iota(jnp.int32, LANES)
    hi_mask = (i16 + LANES) < OUT
    zro = x0_v[pl.ds(0, LANES)] >> 5

    def wrow(k):
        lo = par_v[pl.ds(WOFF + k * OUT, LANES)]
        hi = plsc.load_gather(
            par_v, [zro + i16 + (WOFF + k * OUT + LANES)], mask=hi_mask
        )
        return lo, hi

    w_lo = []
    w_hi = []
    for k in range(2 * EMB):
        lo, hi = wrow(k)
        w_lo.append(lo)
        w_hi.append(hi)

    def escal(row, k):
        return plsc.load_gather(par_v, [zro + (row * EMB + k)])

    for a in (sid, sid + NS):
        a_ok = a < VOCAB

        @pl.when(a_ok)
        def _build(a=a):
            t1_lo = jnp.zeros((LANES,), jnp.float32)
            t1_hi = jnp.zeros((LANES,), jnp.float32)
            for k in range(EMB):
                e = escal(a, k)
                ef = e.astype(jnp.float32)
                t1_lo = t1_lo + ef * w_lo[k]
                t1_hi = t1_hi + ef * w_hi[k]
            for b in range(VOCAB):
                r_lo = t1_lo
                r_hi = t1_hi
                for k in range(EMB):
                    e = escal(b, k)
                    ef = e.astype(jnp.float32)
                    r_lo = r_lo + ef * w_lo[EMB + k]
                    r_hi = r_hi + ef * w_hi[EMB + k]
                tabloc_v[b + 1, pl.ds(0, LANES)] = r_lo
                tabloc_v[b + 1, pl.ds(LANES, LANES)] = r_hi
            pltpu.sync_copy(
                tabloc_v.at[pl.ds(1, VOCAB), :],
                tab_s.at[pl.ds(a * VOCAB, VOCAB), :],
            )

    for j in range(B_PER_W // LANES):
        o = j * LANES
        idx_v[pl.ds(o, LANES)] = (
            x0_v[pl.ds(o, LANES)] * VOCAB + x1_v[pl.ds(o, LANES)]
        )
    plsc.subcore_barrier()
    gathers = []
    for c in range(NCHUNK):
        gathers.append(
            pltpu.async_copy(
                tab_s.at[idx_v.at[pl.ds(c * CHUNK, CHUNK)]],
                rows_v.at[pl.ds(c * CHUNK, CHUNK)],
                gsem,
            )
        )
    writes = []
    for c in range(NCHUNK):
        gathers[c].wait()
        writes.append(
            pltpu.async_copy(
                rows_v.at[pl.ds(c * CHUNK, CHUNK)],
                out_hbm.at[pl.ds(base + c * CHUNK, CHUNK)],
                wsem,
            )
        )
    for w in writes:
        w.wait()


@functools.lru_cache(maxsize=None)
def _make_kernel():
    return pl.kernel(
        _sc_body,
        out_type=jax.ShapeDtypeStruct((BATCH, PAD), jnp.float32),
        mesh=plsc.VectorSubcoreMesh(core_axis_name="c", subcore_axis_name="s"),
        compiler_params=pltpu.CompilerParams(
            needs_layout_passes=False, use_tc_tiling_on_sc=False
        ),
        scratch_types=[
            pltpu.VMEM((PAR_PAD,), jnp.float32),
            pltpu.VMEM((B_PER_W,), jnp.int32),
            pltpu.VMEM((B_PER_W,), jnp.int32),
            pltpu.VMEM((B_PER_W,), jnp.int32),
            pltpu.VMEM((VOCAB + 1, PAD), jnp.float32),
            pltpu.VMEM((B_PER_W, PAD), jnp.float32),
            pltpu.VMEM_SHARED((ROWS, PAD), jnp.float32),
            pltpu.SemaphoreType.DMA,
            pltpu.SemaphoreType.DMA,
        ],
    )


def kernel(xs, embedding, W):
    par = jnp.concatenate([
        embedding.reshape(-1),
        W.reshape(-1),
        jnp.zeros((PAR_PAD - VOCAB * EMB - 2 * EMB * OUT,), jnp.float32),
    ])
    return _make_kernel()(par, xs.T)[:, :OUT]


# speedup vs baseline: 1.2467x; 1.0037x over previous
import functools

import jax
import jax.numpy as jnp
from jax import lax
from jax.experimental import pallas as pl
from jax.experimental.pallas import tpu as pltpu
from jax.experimental.pallas import tpu_sc as plsc

VOCAB = 27
EMB = 5
OUT = 27
PAD = 32
ROWS = 736
BATCH = 16384

NC = 2
NS = 16
NW = NC * NS
B_PER_W = BATCH // NW
CHUNK = 64
NCHUNK = B_PER_W // CHUNK
LANES = 16
PAR_PAD = 408


def _sc_body(par_hbm, xst_hbm, out_hbm,
             par_v, x0_v, x1_v, idx_v, tabloc_v, rows_v, tab_s,
             gsem, wsem):
    sid = lax.axis_index("s")
    wid = sid * NC + lax.axis_index("c")
    base = wid * B_PER_W

    x0c = pltpu.async_copy(xst_hbm.at[0, pl.ds(base, B_PER_W)], x0_v, gsem)
    x1c = pltpu.async_copy(xst_hbm.at[1, pl.ds(base, B_PER_W)], x1_v, gsem)
    pltpu.sync_copy(par_hbm, par_v)
    x0c.wait()
    x1c.wait()
    WOFF = VOCAB * EMB

    i16 = lax.iota(jnp.int32, LANES)
    hi_mask = (i16 + LANES) < OUT
    zro = x0_v[pl.ds(0, LANES)] >> 5

    def wrow(k):
        lo = par_v[pl.ds(WOFF + k * OUT, LANES)]
        hi = plsc.load_gather(
            par_v, [zro + i16 + (WOFF + k * OUT + LANES)], mask=hi_mask
        )
        return lo, hi

    w_lo = []
    w_hi = []
    for k in range(2 * EMB):
        lo, hi = wrow(k)
        w_lo.append(lo)
        w_hi.append(hi)

    def escal(row, k):
        return plsc.load_gather(par_v, [zro + (row * EMB + k)])

    for a in (sid, sid + NS):
        a_ok = a < VOCAB

        @pl.when(a_ok)
        def _build(a=a):
            t1_lo = jnp.zeros((LANES,), jnp.float32)
            t1_hi = jnp.zeros((LANES,), jnp.float32)
            for k in range(EMB):
                e = escal(a, k)
                ef = e.astype(jnp.float32)
                t1_lo = t1_lo + ef * w_lo[k]
                t1_hi = t1_hi + ef * w_hi[k]
            for b in range(VOCAB):
                r_lo = t1_lo
                r_hi = t1_hi
                for k in range(EMB):
                    e = escal(b, k)
                    ef = e.astype(jnp.float32)
                    r_lo = r_lo + ef * w_lo[EMB + k]
                    r_hi = r_hi + ef * w_hi[EMB + k]
                tabloc_v[b + 1, pl.ds(0, LANES)] = r_lo
                tabloc_v[b + 1, pl.ds(LANES, LANES)] = r_hi
            pltpu.sync_copy(
                tabloc_v.at[pl.ds(1, VOCAB), :],
                tab_s.at[pl.ds(a * VOCAB, VOCAB), :],
            )

    for j in range(B_PER_W // LANES):
        o = j * LANES
        idx_v[pl.ds(o, LANES)] = (
            x0_v[pl.ds(o, LANES)] * VOCAB + x1_v[pl.ds(o, LANES)]
        )
    plsc.subcore_barrier()
    gathers = []
    for c in range(NCHUNK):
        gathers.append(
            pltpu.async_copy(
                tab_s.at[idx_v.at[pl.ds(c * CHUNK, CHUNK)]],
                rows_v.at[pl.ds(c * CHUNK, CHUNK)],
                gsem,
            )
        )
    writes = []
    for c in range(NCHUNK):
        gathers[c].wait()
        writes.append(
            pltpu.async_copy(
                rows_v.at[pl.ds(c * CHUNK, CHUNK)],
                out_hbm.at[pl.ds(base + c * CHUNK, CHUNK)],
                wsem,
            )
        )
    for w in writes:
        w.wait()


@functools.lru_cache(maxsize=None)
def _make_kernel():
    return pl.kernel(
        _sc_body,
        out_type=jax.ShapeDtypeStruct((BATCH, PAD), jnp.float32),
        mesh=plsc.VectorSubcoreMesh(core_axis_name="c", subcore_axis_name="s"),
        compiler_params=pltpu.CompilerParams(
            needs_layout_passes=False, use_tc_tiling_on_sc=False
        ),
        scratch_types=[
            pltpu.VMEM((PAR_PAD,), jnp.float32),
            pltpu.VMEM((B_PER_W,), jnp.int32),
            pltpu.VMEM((B_PER_W,), jnp.int32),
            pltpu.VMEM((B_PER_W,), jnp.int32),
            pltpu.VMEM((VOCAB + 1, PAD), jnp.float32),
            pltpu.VMEM((B_PER_W, PAD), jnp.float32),
            pltpu.VMEM_SHARED((ROWS, PAD), jnp.float32),
            pltpu.SemaphoreType.DMA,
            pltpu.SemaphoreType.DMA,
        ],
    )


def kernel(xs, embedding, W):
    par = jnp.concatenate([
        embedding.reshape(-1),
        W.reshape(-1),
        jnp.zeros((PAR_PAD - VOCAB * EMB - 2 * EMB * OUT,), jnp.float32),
    ])
    return _make_kernel()(par, xs.T)[:, :OUT]


# speedup vs baseline: 1.2499x; 1.0026x over previous
import functools

import jax
import jax.numpy as jnp
from jax import lax
from jax.experimental import pallas as pl
from jax.experimental.pallas import tpu as pltpu
from jax.experimental.pallas import tpu_sc as plsc

VOCAB = 27
EMB = 5
OUT = 27
PAD = 32
ROWS = 736
BATCH = 16384

NC = 2
NS = 16
NW = NC * NS
B_PER_W = BATCH // NW
CHUNK = 128
NCHUNK = B_PER_W // CHUNK
LANES = 16
PAR_PAD = 408


def _sc_body(par_hbm, xst_hbm, out_hbm,
             par_v, x0_v, x1_v, idx_v, tabloc_v, rows_v, tab_s,
             gsem, wsem):
    sid = lax.axis_index("s")
    wid = sid * NC + lax.axis_index("c")
    base = wid * B_PER_W

    x0c = pltpu.async_copy(xst_hbm.at[0, pl.ds(base, B_PER_W)], x0_v, gsem)
    x1c = pltpu.async_copy(xst_hbm.at[1, pl.ds(base, B_PER_W)], x1_v, gsem)
    pltpu.sync_copy(par_hbm, par_v)
    x0c.wait()
    x1c.wait()
    WOFF = VOCAB * EMB

    i16 = lax.iota(jnp.int32, LANES)
    hi_mask = (i16 + LANES) < OUT
    zro = x0_v[pl.ds(0, LANES)] >> 5

    def wrow(k):
        lo = par_v[pl.ds(WOFF + k * OUT, LANES)]
        hi = plsc.load_gather(
            par_v, [zro + i16 + (WOFF + k * OUT + LANES)], mask=hi_mask
        )
        return lo, hi

    w_lo = []
    w_hi = []
    for k in range(2 * EMB):
        lo, hi = wrow(k)
        w_lo.append(lo)
        w_hi.append(hi)

    def escal(row, k):
        return plsc.load_gather(par_v, [zro + (row * EMB + k)])

    for a in (sid, sid + NS):
        a_ok = a < VOCAB

        @pl.when(a_ok)
        def _build(a=a):
            t1_lo = jnp.zeros((LANES,), jnp.float32)
            t1_hi = jnp.zeros((LANES,), jnp.float32)
            for k in range(EMB):
                e = escal(a, k)
                ef = e.astype(jnp.float32)
                t1_lo = t1_lo + ef * w_lo[k]
                t1_hi = t1_hi + ef * w_hi[k]
            for b in range(VOCAB):
                r_lo = t1_lo
                r_hi = t1_hi
                for k in range(EMB):
                    e = escal(b, k)
                    ef = e.astype(jnp.float32)
                    r_lo = r_lo + ef * w_lo[EMB + k]
                    r_hi = r_hi + ef * w_hi[EMB + k]
                tabloc_v[b + 1, pl.ds(0, LANES)] = r_lo
                tabloc_v[b + 1, pl.ds(LANES, LANES)] = r_hi
            pltpu.sync_copy(
                tabloc_v.at[pl.ds(1, VOCAB), :],
                tab_s.at[pl.ds(a * VOCAB, VOCAB), :],
            )

    for j in range(B_PER_W // LANES):
        o = j * LANES
        idx_v[pl.ds(o, LANES)] = (
            x0_v[pl.ds(o, LANES)] * VOCAB + x1_v[pl.ds(o, LANES)]
        )
    plsc.subcore_barrier()
    gathers = []
    for c in range(NCHUNK):
        gathers.append(
            pltpu.async_copy(
                tab_s.at[idx_v.at[pl.ds(c * CHUNK, CHUNK)]],
                rows_v.at[pl.ds(c * CHUNK, CHUNK)],
                gsem,
            )
        )
    writes = []
    for c in range(NCHUNK):
        gathers[c].wait()
        writes.append(
            pltpu.async_copy(
                rows_v.at[pl.ds(c * CHUNK, CHUNK)],
                out_hbm.at[pl.ds(base + c * CHUNK, CHUNK)],
                wsem,
            )
        )
    for w in writes:
        w.wait()


@functools.lru_cache(maxsize=None)
def _make_kernel():
    return pl.kernel(
        _sc_body,
        out_type=jax.ShapeDtypeStruct((BATCH, PAD), jnp.float32),
        mesh=plsc.VectorSubcoreMesh(core_axis_name="c", subcore_axis_name="s"),
        compiler_params=pltpu.CompilerParams(
            needs_layout_passes=False, use_tc_tiling_on_sc=False
        ),
        scratch_types=[
            pltpu.VMEM((PAR_PAD,), jnp.float32),
            pltpu.VMEM((B_PER_W,), jnp.int32),
            pltpu.VMEM((B_PER_W,), jnp.int32),
            pltpu.VMEM((B_PER_W,), jnp.int32),
            pltpu.VMEM((VOCAB + 1, PAD), jnp.float32),
            pltpu.VMEM((B_PER_W, PAD), jnp.float32),
            pltpu.VMEM_SHARED((ROWS, PAD), jnp.float32),
            pltpu.SemaphoreType.DMA,
            pltpu.SemaphoreType.DMA,
        ],
    )


def kernel(xs, embedding, W):
    par = jnp.concatenate([
        embedding.reshape(-1),
        W.reshape(-1),
        jnp.zeros((PAR_PAD - VOCAB * EMB - 2 * EMB * OUT,), jnp.float32),
    ])
    return _make_kernel()(par, xs.T)[:, :OUT]
